# baseline (speedup 1.0000x reference)
import functools

import jax
import jax.numpy as jnp
from jax import lax
from jax.experimental import pallas as pl
from jax.experimental.pallas import tpu as pltpu
from jax.experimental.pallas import tpu_sc as plsc

BATCH = 16384
MAX_LEN = 150
GRID = 4
BLOCK_COLS = BATCH // GRID
NW = 32
SC_COLS = BATCH // NW


def _tc_body(x_ref, ids_ref, mask_ref):
    x = x_ref[...]
    ids_ref[...] = x
    mask_ref[...] = jnp.where(x == 0, 0, 1).astype(jnp.int32)


def _sc_zero_body(out_hbm, zbuf):
    wid = lax.axis_index("s") * 2 + lax.axis_index("c")

    def zrow(r, _):
        for j in range(SC_COLS // 16):
            zbuf[r, pl.ds(j * 16, 16)] = jnp.zeros((16,), jnp.int32)
        return 0

    lax.fori_loop(0, MAX_LEN, zrow, 0, unroll=2)
    pltpu.sync_copy(zbuf, out_hbm.at[:, pl.ds(wid * SC_COLS, SC_COLS)])


def kernel(inputs):
    xt = inputs.T
    spec = pl.BlockSpec((MAX_LEN, BLOCK_COLS), lambda i: (0, i))
    out_shape = jax.ShapeDtypeStruct((MAX_LEN, BATCH), jnp.int32)
    ids, mask = pl.pallas_call(
        _tc_body,
        grid=(GRID,),
        in_specs=[spec],
        out_specs=[spec, spec],
        out_shape=[out_shape, out_shape],
        compiler_params=pltpu.CompilerParams(
            dimension_semantics=("arbitrary",),
        ),
    )(xt)
    mesh = plsc.VectorSubcoreMesh(core_axis_name="c", subcore_axis_name="s")
    type_ids = functools.partial(
        pl.kernel,
        mesh=mesh,
        out_type=out_shape,
        scratch_types=[pltpu.VMEM((MAX_LEN, SC_COLS), jnp.int32)],
    )(_sc_zero_body)()
    return (ids.T, mask.T, type_ids.T)


# speedup vs baseline: 2.0537x; 2.0537x over previous
import jax
import jax.numpy as jnp
from jax.experimental import pallas as pl
from jax.experimental.pallas import tpu as pltpu

BATCH = 16384
MAX_LEN = 150
GRID = 4
BLOCK_COLS = BATCH // GRID


def _body(x_ref, ids_ref, mask_ref, type_ref):
    x = x_ref[...]
    ids_ref[...] = x
    mask_ref[...] = jnp.where(x == 0, 0, 1).astype(jnp.int32)
    type_ref[...] = jnp.zeros_like(x)


def kernel(inputs):
    xt = inputs.T
    spec = pl.BlockSpec((MAX_LEN, BLOCK_COLS), lambda i: (0, i))
    out_shape = jax.ShapeDtypeStruct((MAX_LEN, BATCH), jnp.int32)
    ids, mask, type_ids = pl.pallas_call(
        _body,
        grid=(GRID,),
        in_specs=[spec],
        out_specs=[spec, spec, spec],
        out_shape=[out_shape, out_shape, out_shape],
        compiler_params=pltpu.CompilerParams(
            dimension_semantics=("parallel",),
        ),
    )(xt)
    return (ids.T, mask.T, type_ids.T)


# speedup vs baseline: 2.2223x; 1.0821x over previous
import jax
import jax.numpy as jnp
from jax.experimental import pallas as pl
from jax.experimental.pallas import tpu as pltpu

BATCH = 16384
MAX_LEN = 150
NCH = 4
C = BATCH // NCH
NBUF = 3


def _body(in_hbm, ids_hbm, mask_hbm, type_hbm,
          ibuf, mbuf, zbuf, in_sem, ids_sem, mask_sem, z_sem):
    def in_dma(i, s):
        return pltpu.make_async_copy(
            in_hbm.at[:, pl.ds(i * C, C)], ibuf.at[s], in_sem.at[s])

    def ids_dma(i, s):
        return pltpu.make_async_copy(
            ibuf.at[s], ids_hbm.at[:, pl.ds(i * C, C)], ids_sem.at[s])

    def mask_dma(i, s):
        return pltpu.make_async_copy(
            mbuf.at[s], mask_hbm.at[:, pl.ds(i * C, C)], mask_sem.at[s])

    def z_dma(i):
        return pltpu.make_async_copy(
            zbuf, type_hbm.at[:, pl.ds(i * C, C)], z_sem.at[i])

    in_dma(0, 0).start()
    in_dma(1, 1).start()
    zbuf[...] = jnp.zeros_like(zbuf)
    for i in range(NCH):
        z_dma(i).start()
    for i in range(NCH):
        s = i % NBUF
        j = i + 2
        if j < NCH:
            sp = j % NBUF
            if i >= 1:
                ids_dma(i - 1, sp).wait()
            in_dma(j, sp).start()
        in_dma(i, s).wait()
        ids_dma(i, s).start()
        if i >= NBUF:
            mask_dma(i - NBUF, s).wait()
        mbuf[s] = jnp.where(ibuf[s] == 0, 0, 1).astype(jnp.int32)
        mask_dma(i, s).start()
    for i in range(1, NCH):
        ids_dma(i, i % NBUF).wait()
    for i in range(max(0, NCH - NBUF), NCH):
        mask_dma(i, i % NBUF).wait()
    for i in range(NCH):
        z_dma(i).wait()


def kernel(inputs):
    xt = inputs.T
    out_shape = jax.ShapeDtypeStruct((MAX_LEN, BATCH), jnp.int32)
    any_spec = pl.BlockSpec(memory_space=pl.ANY)
    ids, mask, type_ids = pl.pallas_call(
        _body,
        in_specs=[any_spec],
        out_specs=[any_spec, any_spec, any_spec],
        out_shape=[out_shape, out_shape, out_shape],
        scratch_shapes=[
            pltpu.VMEM((NBUF, MAX_LEN, C), jnp.int32),
            pltpu.VMEM((NBUF, MAX_LEN, C), jnp.int32),
            pltpu.VMEM((MAX_LEN, C), jnp.int32),
            pltpu.SemaphoreType.DMA((NBUF,)),
            pltpu.SemaphoreType.DMA((NBUF,)),
            pltpu.SemaphoreType.DMA((NBUF,)),
            pltpu.SemaphoreType.DMA((NCH,)),
        ],
    )(xt)
    return (ids.T, mask.T, type_ids.T)
